# Initial kernel scaffold; baseline (speedup 1.0000x reference)
import jax
import jax.numpy as jnp
from jax.experimental import pallas as pl


def kernel(inputs, pad_embeddings, post_pad_embeddings):
    raise NotImplementedError("write your pallas kernel here")


# speedup vs baseline: 1.0111x; 1.0111x over previous
import functools

import jax
import jax.numpy as jnp
from jax import lax
from jax.experimental import pallas as pl
from jax.experimental.pallas import tpu as pltpu
from jax.experimental.pallas import tpu_sc as plsc

D = 64
SCALE = 8.0
NC, NS = 2, 16
NW = NC * NS
N_ROWS = 16384 * 50
PER_W = N_ROWS // NW
GATHER_ROWS = 128
KSUB = 4
CHUNK = KSUB * GATHER_ROWS
NCHUNK = PER_W // CHUNK


def _splat(v, j):
    idx = jnp.full((16, 1), j, dtype=jnp.int32)
    return lax.gather(
        v,
        idx,
        lax.GatherDimensionNumbers(
            offset_dims=(), collapsed_slice_dims=(0,), start_index_map=(0,)
        ),
        (1,),
        mode=lax.GatherScatterMode.PROMISE_IN_BOUNDS,
    )


@functools.partial(
    pl.kernel,
    out_type=jax.ShapeDtypeStruct((N_ROWS, D), jnp.float32),
    mesh=plsc.VectorSubcoreMesh(core_axis_name="c", subcore_axis_name="s"),
    scratch_types=[
        pltpu.VMEM((KSUB, GATHER_ROWS), jnp.int32),
        pltpu.VMEM((KSUB, GATHER_ROWS), jnp.int32),
        pltpu.VMEM((CHUNK, D), jnp.float32),
        pltpu.SemaphoreType.DMA,
    ],
    compiler_params=pltpu.CompilerParams(use_tc_tiling_on_sc=False),
)
def _emb_lookup(idx_hbm, table_hbm, out_hbm, idx_v, adj_v, rows_v, sem):
    wid = lax.axis_index("s") * NC + lax.axis_index("c")

    @pl.loop(0, NCHUNK)
    def _chunk(ck):
        base = wid * PER_W + ck * CHUNK
        pltpu.sync_copy(idx_hbm.at[wid, ck], idx_v)

        for k in range(KSUB):
            for c in range(GATHER_ROWS // 16):
                iv = idx_v[k, pl.ds(c * 16, 16)]
                adj_v[k, pl.ds(c * 16, 16)] = jnp.maximum(iv - 1, 0)

        copies = [
            pltpu.async_copy(
                table_hbm.at[adj_v.at[k]],
                rows_v.at[pl.ds(k * GATHER_ROWS, GATHER_ROWS)],
                sem,
            )
            for k in range(KSUB)
        ]
        for cp in copies:
            cp.wait()

        for k in range(KSUB):

            @pl.loop(0, GATHER_ROWS // 16)
            def _scale(c, k=k):
                iv = idx_v[k, pl.ds(c * 16, 16)]
                sv = jnp.where(iv == 0, jnp.float32(0.0), jnp.float32(SCALE))
                rbase = k * GATHER_ROWS + c * 16
                for j in range(16):
                    s = _splat(sv, j)
                    r = rbase + j
                    for q in range(D // 16):
                        rows_v[r, pl.ds(q * 16, 16)] = (
                            rows_v[r, pl.ds(q * 16, 16)] * s
                        )

        pltpu.sync_copy(rows_v, out_hbm.at[pl.ds(base, CHUNK)])


def kernel(inputs, pad_embeddings, post_pad_embeddings):
    del pad_embeddings
    idx = inputs.astype(jnp.int32).reshape(NW, NCHUNK, KSUB, GATHER_ROWS)
    out = _emb_lookup(idx, post_pad_embeddings)
    return out.reshape(inputs.shape[0], inputs.shape[1], D)


# speedup vs baseline: 1.1303x; 1.1179x over previous
import functools

import jax
import jax.numpy as jnp
from jax import lax
from jax.experimental import pallas as pl
from jax.experimental.pallas import tpu as pltpu
from jax.experimental.pallas import tpu_sc as plsc

D = 64
SCALE = 8.0
NC, NS = 2, 16
NW = NC * NS
B = 16384
T = 50
CBLK = 128
NCB = B // CBLK
CB_PER_W = NCB // NW


@functools.partial(
    pl.kernel,
    out_type=jax.ShapeDtypeStruct((T, D // 8, NCB, 8, CBLK), jnp.float32),
    mesh=plsc.VectorSubcoreMesh(core_axis_name="c", subcore_axis_name="s"),
    scratch_types=[
        pltpu.VMEM((CB_PER_W, T, CBLK), jnp.int32),
        pltpu.VMEM((2, CBLK), jnp.int32),
        pltpu.VMEM((2, CBLK, D), jnp.float32),
        pltpu.VMEM((2, D // 8, 8, CBLK), jnp.float32),
        pltpu.SemaphoreType.DMA,
        pltpu.SemaphoreType.DMA,
        pltpu.SemaphoreType.DMA,
        pltpu.SemaphoreType.DMA,
    ],
    compiler_params=pltpu.CompilerParams(
        use_tc_tiling_on_sc=False, needs_layout_passes=False
    ),
)
def _emb_lookup(
    idxt_hbm, table_hbm, out_hbm, idx_v, adj_v, rows_v, trans_v, si0, si1, so0, so1
):
    wid = lax.axis_index("s") * NC + lax.axis_index("c")
    sem_in = (si0, si1)
    sem_out = (so0, so1)

    for cb in range(CB_PER_W):
        pltpu.sync_copy(
            idxt_hbm.at[:, pl.ds((wid * CB_PER_W + cb) * CBLK, CBLK)],
            idx_v.at[cb],
        )

    def prep_adj(cb, t, bb):
        for i0 in range(0, CBLK, 16):
            iv = idx_v[cb, t, pl.ds(i0, 16)]
            adj_v[bb, pl.ds(i0, 16)] = jnp.maximum(iv - 1, 0)

    def fire_gather(bb):
        pltpu.async_copy(table_hbm.at[adj_v.at[bb]], rows_v.at[bb], sem_in[bb])

    def wait_gather(bb):
        pltpu.make_async_copy(
            table_hbm.at[adj_v.at[bb]], rows_v.at[bb], sem_in[bb]
        ).wait()

    def drain_out(bb):
        pltpu.make_async_copy(out_hbm.at[0, :, 0], trans_v.at[bb], sem_out[bb]).wait()

    def compute(cb, t, bb):
        for i0 in range(0, CBLK, 16):
            iv = idx_v[cb, t, pl.ds(i0, 16)]
            sv = jnp.where(iv == 0, jnp.float32(0.0), jnp.float32(SCALE))
            rows16 = lax.iota(jnp.int32, 16) + i0
            for c0 in range(0, D, 8):
                gs = [
                    plsc.load_gather(
                        rows_v.at[bb],
                        [rows16, jnp.full((16,), c0 + k, jnp.int32)],
                    )
                    for k in range(8)
                ]
                for k in range(8):
                    trans_v[bb, c0 // 8, k, pl.ds(i0, 16)] = gs[k] * sv

    def fire_out(t, c_glob, bb):
        pltpu.async_copy(trans_v.at[bb], out_hbm.at[t, :, c_glob], sem_out[bb])

    @pl.loop(0, CB_PER_W)
    def _blk(cb):
        c_glob = wid * CB_PER_W + cb
        prep_adj(cb, 0, 0)
        fire_gather(0)

        @pl.loop(0, T, step=2)
        def _t(t):
            for phase in range(2):
                tp = t + phase
                bb = phase
                if phase == 0:
                    prep_adj(cb, tp + 1, 1)
                    fire_gather(1)
                else:

                    @pl.when(tp + 1 < T)
                    def _():
                        prep_adj(cb, tp + 1, 0)
                        fire_gather(0)

                wait_gather(bb)

                @pl.when(cb * T + tp >= 2)
                def _():
                    drain_out(bb)

                compute(cb, tp, bb)
                fire_out(tp, c_glob, bb)

    drain_out(0)
    drain_out(1)


def kernel(inputs, pad_embeddings, post_pad_embeddings):
    del pad_embeddings
    idxt = inputs.astype(jnp.int32).T
    out5 = _emb_lookup(idxt, post_pad_embeddings)
    return out5.transpose(2, 4, 0, 1, 3).reshape(B, T, D)


# speedup vs baseline: 1.1599x; 1.0262x over previous
import functools

import jax
import jax.numpy as jnp
from jax import lax
from jax.experimental import pallas as pl
from jax.experimental.pallas import tpu as pltpu
from jax.experimental.pallas import tpu_sc as plsc

D = 64
SCALE = 8.0
NC, NS = 2, 16
NW = NC * NS
B = 16384
T = 50
CBLK = 128
NCB = B // CBLK
CB_PER_W = NCB // NW
NUNITS = CB_PER_W * T


@functools.partial(
    pl.kernel,
    out_type=jax.ShapeDtypeStruct((T, D // 8, NCB, 8, CBLK), jnp.float32),
    mesh=plsc.VectorSubcoreMesh(core_axis_name="c", subcore_axis_name="s"),
    scratch_types=[
        pltpu.VMEM((CB_PER_W, T, CBLK), jnp.int32),
        pltpu.VMEM((4, CBLK), jnp.int32),
        pltpu.VMEM((4, CBLK, D), jnp.float32),
        pltpu.VMEM((2, D // 8, 8, CBLK), jnp.float32),
        pltpu.SemaphoreType.DMA,
        pltpu.SemaphoreType.DMA,
        pltpu.SemaphoreType.DMA,
        pltpu.SemaphoreType.DMA,
        pltpu.SemaphoreType.DMA,
        pltpu.SemaphoreType.DMA,
    ],
    compiler_params=pltpu.CompilerParams(
        use_tc_tiling_on_sc=False, needs_layout_passes=False
    ),
)
def _emb_lookup(
    idxt_hbm, table_hbm, out_hbm, idx_v, adj_v, rows_v, trans_v,
    si0, si1, si2, si3, so0, so1,
):
    wid = lax.axis_index("s") * NC + lax.axis_index("c")
    sem_in = (si0, si1, si2, si3)
    sem_out = (so0, so1)

    for cb in range(CB_PER_W):
        pltpu.sync_copy(
            idxt_hbm.at[:, pl.ds((wid * CB_PER_W + cb) * CBLK, CBLK)],
            idx_v.at[cb],
        )

    def prep_and_fire(cb, t, bb):
        for i0 in range(0, CBLK, 16):
            iv = idx_v[cb, t, pl.ds(i0, 16)]
            adj_v[bb, pl.ds(i0, 16)] = jnp.maximum(iv - 1, 0)
        pltpu.async_copy(table_hbm.at[adj_v.at[bb]], rows_v.at[bb], sem_in[bb])

    def wait_gather(bb):
        pltpu.make_async_copy(
            table_hbm.at[adj_v.at[bb]], rows_v.at[bb], sem_in[bb]
        ).wait()

    def drain_out(tb):
        pltpu.make_async_copy(out_hbm.at[0, :, 0], trans_v.at[tb], sem_out[tb]).wait()

    def compute(cb, t, bb, tb):
        @pl.loop(0, CBLK, step=16)
        def _i0(i0):
            iv = idx_v[cb, t, pl.ds(i0, 16)]
            sv = jnp.where(iv == 0, jnp.float32(0.0), jnp.float32(SCALE))
            rows16 = lax.iota(jnp.int32, 16) + i0
            for c0 in range(0, D, 8):
                gs = [
                    plsc.load_gather(
                        rows_v.at[bb],
                        [rows16, jnp.full((16,), c0 + k, jnp.int32)],
                    )
                    for k in range(8)
                ]
                for k in range(8):
                    trans_v[tb, c0 // 8, k, pl.ds(i0, 16)] = gs[k] * sv

    for v in range(3):
        prep_and_fire(v & 3, v >> 2, v & 3)

    @pl.loop(0, NUNITS, step=4)
    def _u0(u0):
        for phase in range(4):
            u = u0 + phase
            cb = phase
            bb = phase
            tb = phase & 1
            t = u >> 2

            nxt = u + 3

            @pl.when(nxt < NUNITS)
            def _():
                prep_and_fire((phase + 3) & 3, nxt >> 2, (phase + 3) & 3)

            wait_gather(bb)

            @pl.when(u >= 2)
            def _():
                drain_out(tb)

            compute(cb, t, bb, tb)
            pltpu.async_copy(
                trans_v.at[tb], out_hbm.at[t, :, wid * CB_PER_W + cb], sem_out[tb]
            )

    drain_out(0)
    drain_out(1)


def kernel(inputs, pad_embeddings, post_pad_embeddings):
    del pad_embeddings
    idxt = inputs.astype(jnp.int32).T
    out5 = _emb_lookup(idxt, post_pad_embeddings)
    return out5.transpose(2, 4, 0, 1, 3).reshape(B, T, D)


# speedup vs baseline: 1.2763x; 1.1004x over previous
import functools

import jax
import jax.numpy as jnp
from jax import lax
from jax.experimental import pallas as pl
from jax.experimental.pallas import tpu as pltpu
from jax.experimental.pallas import tpu_sc as plsc

D = 64
SCALE = 8.0
NC, NS = 2, 16
NW = NC * NS
B = 16384
T = 50
CBLK = 128
NCB = B // CBLK
CB_PER_W = NCB // NW
NUNITS = CB_PER_W * T


@functools.partial(
    pl.kernel,
    out_type=jax.ShapeDtypeStruct((T, D // 8, NCB, 8, CBLK), jnp.float32),
    mesh=plsc.VectorSubcoreMesh(core_axis_name="c", subcore_axis_name="s"),
    scratch_types=[
        pltpu.VMEM((CB_PER_W, T, CBLK), jnp.int32),
        pltpu.VMEM((4, CBLK), jnp.int32),
        pltpu.VMEM((4, CBLK, D), jnp.float32),
        pltpu.VMEM((2, D // 8, 8, CBLK), jnp.float32),
        pltpu.SemaphoreType.DMA,
        pltpu.SemaphoreType.DMA,
        pltpu.SemaphoreType.DMA,
        pltpu.SemaphoreType.DMA,
        pltpu.SemaphoreType.DMA,
        pltpu.SemaphoreType.DMA,
    ],
    compiler_params=pltpu.CompilerParams(
        use_tc_tiling_on_sc=False, needs_layout_passes=False
    ),
)
def _emb_lookup(
    idxt_hbm, table_hbm, out_hbm, idx_v, adj_v, rows_v, trans_v,
    si0, si1, si2, si3, so0, so1,
):
    wid = lax.axis_index("s") * NC + lax.axis_index("c")
    sem_in = (si0, si1, si2, si3)
    sem_out = (so0, so1)

    for cb in range(CB_PER_W):
        pltpu.sync_copy(
            idxt_hbm.at[:, pl.ds((wid * CB_PER_W + cb) * CBLK, CBLK)],
            idx_v.at[cb],
        )

    def prep_and_fire(cb, t, bb):
        for i0 in range(0, CBLK, 16):
            iv = idx_v[cb, t, pl.ds(i0, 16)]
            adj_v[bb, pl.ds(i0, 16)] = jnp.maximum(iv - 1, 0)
        pltpu.async_copy(table_hbm.at[adj_v.at[bb]], rows_v.at[bb], sem_in[bb])

    def wait_gather(bb):
        pltpu.make_async_copy(
            table_hbm.at[adj_v.at[bb]], rows_v.at[bb], sem_in[bb]
        ).wait()

    def drain_out(tb):
        pltpu.make_async_copy(out_hbm.at[0, :, 0], trans_v.at[tb], sem_out[tb]).wait()

    def compute(cb, t, bb, tb):
        @pl.loop(0, CBLK, step=16)
        def _i0(i0):
            iv = idx_v[cb, t, pl.ds(i0, 16)]
            sv = jnp.where(iv == 0, jnp.float32(0.0), jnp.float32(SCALE))
            lanes = lax.iota(jnp.int32, 16)
            rows16 = lanes + i0
            for c0 in range(D):
                cols16 = (lanes + c0) & (D - 1)
                g = plsc.load_gather(rows_v.at[bb], [rows16, cols16])
                plsc.store_scatter(
                    trans_v.at[tb],
                    [cols16 >> 3, cols16 & 7, rows16],
                    g * sv,
                )

    for v in range(3):
        prep_and_fire(v & 3, v >> 2, v & 3)

    @pl.loop(0, NUNITS, step=4)
    def _u0(u0):
        for phase in range(4):
            u = u0 + phase
            cb = phase
            bb = phase
            tb = phase & 1
            t = u >> 2

            nxt = u + 3

            @pl.when(nxt < NUNITS)
            def _():
                prep_and_fire((phase + 3) & 3, nxt >> 2, (phase + 3) & 3)

            wait_gather(bb)

            @pl.when(u >= 2)
            def _():
                drain_out(tb)

            compute(cb, t, bb, tb)
            pltpu.async_copy(
                trans_v.at[tb], out_hbm.at[t, :, wid * CB_PER_W + cb], sem_out[tb]
            )

    drain_out(0)
    drain_out(1)


def kernel(inputs, pad_embeddings, post_pad_embeddings):
    del pad_embeddings
    idxt = inputs.astype(jnp.int32).T
    out5 = _emb_lookup(idxt, post_pad_embeddings)
    return out5.transpose(2, 4, 0, 1, 3).reshape(B, T, D)


# speedup vs baseline: 2.0164x; 1.5798x over previous
import functools

import jax
import jax.numpy as jnp
from jax import lax
from jax.experimental import pallas as pl
from jax.experimental.pallas import tpu as pltpu
from jax.experimental.pallas import tpu_sc as plsc

D = 64
SCALE = 8.0
NC, NS = 2, 16
NW = NC * NS
B = 16384
T = 50
CBLK = 128
NCB = B // CBLK
CB_PER_W = NCB // NW
NUNITS = CB_PER_W * T


@functools.partial(
    pl.kernel,
    out_type=jax.ShapeDtypeStruct((T, D // 8, NCB, 8, CBLK), jnp.float32),
    mesh=plsc.VectorSubcoreMesh(core_axis_name="c", subcore_axis_name="s"),
    scratch_types=[
        pltpu.VMEM((CB_PER_W, T, CBLK), jnp.int32),
        pltpu.VMEM((4, CBLK), jnp.int32),
        pltpu.VMEM((4, CBLK, D), jnp.float32),
        pltpu.VMEM((2, D // 8, 8, CBLK), jnp.float32),
        pltpu.SemaphoreType.DMA,
        pltpu.SemaphoreType.DMA,
        pltpu.SemaphoreType.DMA,
        pltpu.SemaphoreType.DMA,
        pltpu.SemaphoreType.DMA,
        pltpu.SemaphoreType.DMA,
    ],
    compiler_params=pltpu.CompilerParams(
        use_tc_tiling_on_sc=False, needs_layout_passes=False
    ),
)
def _emb_lookup(
    idxt_hbm, table_hbm, out_hbm, idx_v, adj_v, rows_v, trans_v,
    si0, si1, si2, si3, so0, so1,
):
    wid = lax.axis_index("s") * NC + lax.axis_index("c")
    sem_in = (si0, si1, si2, si3)
    sem_out = (so0, so1)

    for cb in range(CB_PER_W):
        pltpu.sync_copy(
            idxt_hbm.at[:, pl.ds((wid * CB_PER_W + cb) * CBLK, CBLK)],
            idx_v.at[cb],
        )

    def prep_and_fire(cb, t, bb):
        for i0 in range(0, CBLK, 16):
            iv = idx_v[cb, t, pl.ds(i0, 16)]
            adj_v[bb, pl.ds(i0, 16)] = jnp.maximum(iv - 1, 0)
        pltpu.async_copy(table_hbm.at[adj_v.at[bb]], rows_v.at[bb], sem_in[bb])

    def wait_gather(bb):
        pltpu.make_async_copy(
            table_hbm.at[adj_v.at[bb]], rows_v.at[bb], sem_in[bb]
        ).wait()

    def drain_out(tb):
        pltpu.make_async_copy(out_hbm.at[0, :, 0], trans_v.at[tb], sem_out[tb]).wait()

    def compute(cb, t, bb, tb):
        @pl.loop(0, CBLK, step=16)
        def _i0(i0):
            iv = idx_v[cb, t, pl.ds(i0, 16)]
            sv = jnp.where(iv == 0, jnp.float32(0.0), jnp.float32(SCALE))
            lanes = lax.iota(jnp.int32, 16)
            rows16 = lanes + i0

            @pl.loop(0, D, step=8)
            def _c(c0):
                base = lanes + c0
                cols = [(base + k) & (D - 1) for k in range(8)]
                gs = [
                    plsc.load_gather(rows_v.at[bb], [rows16, cols[k]])
                    for k in range(8)
                ]
                for k in range(8):
                    plsc.store_scatter(
                        trans_v.at[tb],
                        [cols[k] >> 3, cols[k] & 7, rows16],
                        gs[k] * sv,
                    )

    for v in range(3):
        prep_and_fire(v & 3, v >> 2, v & 3)

    @pl.loop(0, NUNITS, step=4)
    def _u0(u0):
        for phase in range(4):
            u = u0 + phase
            cb = phase
            bb = phase
            tb = phase & 1
            t = u >> 2

            nxt = u + 3

            @pl.when(nxt < NUNITS)
            def _():
                prep_and_fire((phase + 3) & 3, nxt >> 2, (phase + 3) & 3)

            wait_gather(bb)

            @pl.when(u >= 2)
            def _():
                drain_out(tb)

            compute(cb, t, bb, tb)
            pltpu.async_copy(
                trans_v.at[tb], out_hbm.at[t, :, wid * CB_PER_W + cb], sem_out[tb]
            )

    drain_out(0)
    drain_out(1)


def kernel(inputs, pad_embeddings, post_pad_embeddings):
    del pad_embeddings
    idxt = inputs.astype(jnp.int32).T
    out5 = _emb_lookup(idxt, post_pad_embeddings)
    return out5.transpose(2, 4, 0, 1, 3).reshape(B, T, D)
